# baseline (speedup 1.0000x reference)
import functools
import math

import jax
import jax.numpy as jnp
from jax import lax
from jax.experimental import pallas as pl
from jax.experimental.pallas import tpu as pltpu
from jax.experimental.pallas import tpu_sc as plsc

D_MODEL = 128
SCALE = math.sqrt(D_MODEL)
LANES = 16
NUM_CORES = 2
NUM_SUBCORES = 16
NUM_WORKERS = NUM_CORES * NUM_SUBCORES

BATCH = 4096
SEQ = 50
COLS_PER_W = BATCH // NUM_WORKERS
NSLOT = 6
AHEAD = 3

_mesh = plsc.VectorSubcoreMesh(core_axis_name="c", subcore_axis_name="s")


@functools.partial(
    pl.kernel,
    mesh=_mesh,
    out_type=jax.ShapeDtypeStruct((SEQ, BATCH, D_MODEL), jnp.float32),
    scratch_types=[
        pltpu.VMEM((SEQ, COLS_PER_W), jnp.int32),
        pltpu.VMEM((NSLOT, COLS_PER_W, D_MODEL), jnp.float32),
        pltpu.SemaphoreType.DMA,
        pltpu.SemaphoreType.DMA,
    ],
)
def _emb_lookup(xt_hbm, table_hbm, out_hbm, idx_v, ring_v, gsem, ssem):
    wid = lax.axis_index("s") * NUM_CORES + lax.axis_index("c")
    b0 = wid * COLS_PER_W

    pltpu.sync_copy(xt_hbm.at[:, pl.ds(b0, COLS_PER_W)], idx_v)

    def start_gather(ss, t):
        pltpu.async_copy(table_hbm.at[idx_v.at[ss]], ring_v.at[t], gsem)

    def wait_gather(t):
        pltpu.make_async_copy(table_hbm.at[idx_v.at[0]], ring_v.at[t], gsem).wait()

    def start_scatter(ss, t):
        pltpu.async_copy(ring_v.at[t], out_hbm.at[ss, pl.ds(b0, COLS_PER_W)], ssem)

    def wait_scatter(t):
        pltpu.make_async_copy(
            ring_v.at[t], out_hbm.at[0, pl.ds(b0, COLS_PER_W)], ssem
        ).wait()

    def compute(ss, slab):
        def group(g, carry):
            r0 = g * LANES
            idx16 = idx_v[ss, pl.ds(r0, LANES)]
            scale16 = jnp.where(idx16 == 0, 0.0, SCALE).astype(jnp.float32)
            for l in range(LANES):
                scale = jnp.full((LANES,), scale16[l], jnp.float32)
                for c in range(D_MODEL // LANES):
                    sl = pl.ds(c * LANES, LANES)
                    slab[r0 + l, sl] = slab[r0 + l, sl] * scale
            return carry

        lax.fori_loop(0, COLS_PER_W // LANES, group, 0)

    for t in range(AHEAD):
        start_gather(t, t)

    def body(ss, carry):
        t = lax.rem(ss, NSLOT)

        @pl.when(ss < SEQ)
        def _():
            wait_gather(t)
            compute(ss, ring_v.at[t])
            start_scatter(ss, t)

        @pl.when(ss >= AHEAD)
        def _():
            wait_scatter(t)

        @pl.when(ss + AHEAD < SEQ)
        def _():
            start_gather(ss + AHEAD, lax.rem(ss + AHEAD, NSLOT))

        return carry

    lax.fori_loop(0, SEQ + AHEAD, body, 0)


def kernel(x, table):
    out_t = _emb_lookup(x.T, table)
    return jnp.transpose(out_t, (1, 0, 2))


# speedup vs baseline: 2.7213x; 2.7213x over previous
import functools
import math

import jax
import jax.numpy as jnp
from jax import lax
from jax.experimental import pallas as pl
from jax.experimental.pallas import tpu as pltpu
from jax.experimental.pallas import tpu_sc as plsc

D_MODEL = 128
SCALE = math.sqrt(D_MODEL)
LANES = 16
NUM_CORES = 2
NUM_SUBCORES = 16
NUM_WORKERS = NUM_CORES * NUM_SUBCORES

BATCH = 4096
SEQ = 50
COLS_PER_W = BATCH // NUM_WORKERS
NSLOT = 6
AHEAD = 3

_mesh = plsc.VectorSubcoreMesh(core_axis_name="c", subcore_axis_name="s")


@functools.partial(
    pl.kernel,
    mesh=_mesh,
    out_type=jax.ShapeDtypeStruct((SEQ, BATCH, D_MODEL), jnp.float32),
    scratch_types=[
        pltpu.VMEM((SEQ, COLS_PER_W), jnp.int32),
        pltpu.VMEM((NSLOT, COLS_PER_W, D_MODEL), jnp.float32),
        pltpu.SemaphoreType.DMA,
        pltpu.SemaphoreType.DMA,
    ],
)
def _emb_lookup(xt_hbm, table_hbm, out_hbm, idx_v, ring_v, gsem, ssem):
    wid = lax.axis_index("s") * NUM_CORES + lax.axis_index("c")
    b0 = wid * COLS_PER_W

    pltpu.sync_copy(xt_hbm.at[:, pl.ds(b0, COLS_PER_W)], idx_v)

    def start_gather(ss, t):
        pltpu.async_copy(table_hbm.at[idx_v.at[ss]], ring_v.at[t], gsem)

    def wait_gather(t):
        pltpu.make_async_copy(table_hbm.at[idx_v.at[0]], ring_v.at[t], gsem).wait()

    def start_scatter(ss, t):
        pltpu.async_copy(ring_v.at[t], out_hbm.at[ss, pl.ds(b0, COLS_PER_W)], ssem)

    def wait_scatter(t):
        pltpu.make_async_copy(
            ring_v.at[t], out_hbm.at[0, pl.ds(b0, COLS_PER_W)], ssem
        ).wait()

    def compute(ss, slab):
        def group(g, carry):
            r0 = g * LANES
            idx16 = idx_v[ss, pl.ds(r0, LANES)]
            scale16 = jnp.where(idx16 == 0, 0.0, SCALE).astype(jnp.float32)
            for l in range(LANES):
                scale = jnp.full((LANES,), scale16[l], jnp.float32)
                for c in range(D_MODEL // LANES):
                    sl = pl.ds(c * LANES, LANES)
                    slab[r0 + l, sl] = slab[r0 + l, sl] * scale
            return carry

        lax.fori_loop(0, COLS_PER_W // LANES, group, 0)

    def step(ss, t):
        wait_gather(t)
        compute(ss, ring_v.at[t])
        start_scatter(ss, t)

        @pl.when(ss >= AHEAD)
        def _():
            wait_scatter(t)

        @pl.when(ss + AHEAD < SEQ)
        def _():
            start_gather(ss + AHEAD, (t + AHEAD) % NSLOT)

    for t in range(AHEAD):
        start_gather(t, t)

    def body(j, carry):
        for t in range(NSLOT):
            step(j * NSLOT + t, t)
        return carry

    lax.fori_loop(0, (SEQ - 2) // NSLOT, body, 0)
    step(48, 0)
    step(49, 1)
    for _ in range(AHEAD):
        wait_scatter(0)


def kernel(x, table):
    out_t = _emb_lookup(x.T, table)
    return jnp.transpose(out_t, (1, 0, 2))
